# baseline (speedup 1.0000x reference)
import functools

import jax
import jax.numpy as jnp
from jax import lax
from jax.experimental import pallas as pl
from jax.experimental.pallas import tpu as pltpu
from jax.experimental.pallas import tpu_sc as plsc

_K = 1024
_C = 512
_BM = 1024


def _tc_argmin_body(z_ref, e_ref, s2_ref, idx_ref):
    zb = z_ref[0]
    e = e_ref[...]
    s1 = jnp.sum(zb * zb, axis=0, keepdims=True)
    mm = lax.dot_general(e, zb, (((1,), (0,)), ((), ())),
                         preferred_element_type=jnp.float32)
    d = (s1 - 2.0 * mm) + s2_ref[...]
    m = jnp.min(d, axis=0, keepdims=True)
    kiota = lax.broadcasted_iota(jnp.int32, (_K, _BM), 0)
    idx = jnp.min(jnp.where(d == m, kiota, _K), axis=0)
    idx_ref[0, 0, :] = idx


def _tc_argmin(z_col, e, s2):
    grid = z_col.shape[0]
    out = pl.pallas_call(
        _tc_argmin_body,
        grid=(grid,),
        in_specs=[
            pl.BlockSpec((1, _C, _BM), lambda i: (i, 0, 0)),
            pl.BlockSpec((_K, _C), lambda i: (0, 0)),
            pl.BlockSpec((_K, 1), lambda i: (0, 0)),
        ],
        out_specs=pl.BlockSpec((1, 1, _BM), lambda i: (i, 0, 0)),
        out_shape=jax.ShapeDtypeStruct((grid, 1, _BM), jnp.int32),
    )(z_col, e, s2)
    return out.reshape(grid * _BM)


_NW = 32
_CH = 64


def _sc_gather_body(table_hbm, idx_hbm, out_hbm, idx_v, rows_v, sem0, sem1):
    wid = lax.axis_index("s") * 2 + lax.axis_index("c")
    b_per_w = idx_v.shape[0]
    nch = b_per_w // _CH
    base = wid * b_per_w
    pltpu.sync_copy(idx_hbm.at[pl.ds(base, b_per_w)], idx_v)
    sems = (sem0, sem1)

    def start(c):
        return pltpu.async_copy(
            table_hbm.at[idx_v.at[pl.ds(c * _CH, _CH)]],
            rows_v.at[c % 2], sems[c % 2])

    cp = start(0)
    for c in range(nch):
        nxt = start(c + 1) if c + 1 < nch else None
        cp.wait()
        pltpu.sync_copy(rows_v.at[c % 2],
                        out_hbm.at[pl.ds(base + c * _CH, _CH)])
        cp = nxt


def _sc_gather(table, idx):
    n = idx.shape[0]
    b_per_w = n // _NW
    mesh = plsc.VectorSubcoreMesh(core_axis_name="c", subcore_axis_name="s")
    f = functools.partial(
        pl.kernel,
        out_type=jax.ShapeDtypeStruct((n, _C), jnp.float32),
        mesh=mesh,
        scratch_types=[
            pltpu.VMEM((b_per_w,), jnp.int32),
            pltpu.VMEM((2, _CH, _C), jnp.float32),
            pltpu.SemaphoreType.DMA,
            pltpu.SemaphoreType.DMA,
        ],
    )(_sc_gather_body)
    return f(table, idx)


def kernel(z, embedding_weight):
    B, C, H, W = z.shape
    z_col = z.reshape(B, C, H * W)
    s2 = jnp.sum(embedding_weight ** 2, axis=1, keepdims=True)
    idx = _tc_argmin(z_col, embedding_weight, s2)
    quantized = _sc_gather(embedding_weight, idx).reshape(z.shape)
    return (quantized, idx.reshape(B, -1))


# speedup vs baseline: 1.1959x; 1.1959x over previous
import functools

import jax
import jax.numpy as jnp
from jax import lax
from jax.experimental import pallas as pl
from jax.experimental.pallas import tpu as pltpu
from jax.experimental.pallas import tpu_sc as plsc

_K = 1024
_C = 512
_BM = 512


def _tc_argmin_body(z_ref, e_ref, s2_ref, idx_ref):
    zb = z_ref[...]
    e = e_ref[...]
    s1 = jnp.sum(zb * zb, axis=1, keepdims=True)
    mm = lax.dot_general(zb, e, (((1,), (1,)), ((), ())),
                         preferred_element_type=jnp.float32)
    d = (s1 - 2.0 * mm) + s2_ref[...]
    m = jnp.min(d, axis=1, keepdims=True)
    kiota = lax.broadcasted_iota(jnp.int32, (_BM, _K), 1)
    idx = jnp.min(jnp.where(d == m, kiota, _K), axis=1)
    idx_ref[0, 0, :] = idx


def _tc_argmin(z_flat, e, s2t):
    n = z_flat.shape[0]
    grid = n // _BM
    out = pl.pallas_call(
        _tc_argmin_body,
        grid=(grid,),
        in_specs=[
            pl.BlockSpec((_BM, _C), lambda i: (i, 0)),
            pl.BlockSpec((_K, _C), lambda i: (0, 0)),
            pl.BlockSpec((1, _K), lambda i: (0, 0)),
        ],
        out_specs=pl.BlockSpec((1, 1, _BM), lambda i: (i, 0, 0)),
        out_shape=jax.ShapeDtypeStruct((grid, 1, _BM), jnp.int32),
    )(z_flat, e, s2t)
    return out.reshape(n)


_NW = 32
_CH = 64


def _sc_gather_body(table_hbm, idx_hbm, out_hbm, idx_v, rows_v, sem0, sem1):
    wid = lax.axis_index("s") * 2 + lax.axis_index("c")
    b_per_w = idx_v.shape[0]
    nch = b_per_w // _CH
    base = wid * b_per_w
    pltpu.sync_copy(idx_hbm.at[pl.ds(base, b_per_w)], idx_v)
    sems = (sem0, sem1)

    def start(c):
        return pltpu.async_copy(
            table_hbm.at[idx_v.at[pl.ds(c * _CH, _CH)]],
            rows_v.at[c % 2], sems[c % 2])

    cp = start(0)
    for c in range(nch):
        nxt = start(c + 1) if c + 1 < nch else None
        cp.wait()
        pltpu.sync_copy(rows_v.at[c % 2],
                        out_hbm.at[pl.ds(base + c * _CH, _CH)])
        cp = nxt


def _sc_gather(table, idx):
    n = idx.shape[0]
    b_per_w = n // _NW
    mesh = plsc.VectorSubcoreMesh(core_axis_name="c", subcore_axis_name="s")
    f = functools.partial(
        pl.kernel,
        out_type=jax.ShapeDtypeStruct((n, _C), jnp.float32),
        mesh=mesh,
        scratch_types=[
            pltpu.VMEM((b_per_w,), jnp.int32),
            pltpu.VMEM((2, _CH, _C), jnp.float32),
            pltpu.SemaphoreType.DMA,
            pltpu.SemaphoreType.DMA,
        ],
    )(_sc_gather_body)
    return f(table, idx)


_NCHUNK = 4


def kernel(z, embedding_weight):
    B, C, H, W = z.shape
    z_flat = jnp.transpose(z, (0, 2, 3, 1)).reshape(-1, C)
    s2t = jnp.sum(embedding_weight ** 2, axis=1, keepdims=True).T
    bc = B // _NCHUNK
    rows = bc * H * W
    quants, idxs = [], []
    for ci in range(_NCHUNK):
        zc = lax.slice_in_dim(z_flat, ci * rows, (ci + 1) * rows, axis=0)
        idx_c = _tc_argmin(zc, embedding_weight, s2t)
        g_c = _sc_gather(embedding_weight, idx_c)
        gv = g_c.reshape(bc, C, 2, C)
        v = jnp.transpose(gv, (0, 2, 3, 1))
        v = v.reshape(bc, H, W, C)
        quants.append(jnp.transpose(v, (0, 3, 1, 2)))
        idxs.append(idx_c.reshape(bc, -1))
    quantized = jnp.concatenate(quants, axis=0)
    idx = jnp.concatenate(idxs, axis=0)
    return (quantized, idx)


# speedup vs baseline: 1.1982x; 1.0020x over previous
import functools

import jax
import jax.numpy as jnp
from jax import lax
from jax.experimental import pallas as pl
from jax.experimental.pallas import tpu as pltpu
from jax.experimental.pallas import tpu_sc as plsc

_K = 1024
_C = 512
_BM = 512


_KT = 256


def _tc_argmin_body(z_ref, e_ref, s2_ref, idx_ref):
    zb = z_ref[...]
    s1 = jnp.sum(zb * zb, axis=1, keepdims=True)
    tiota = lax.broadcasted_iota(jnp.int32, (_BM, _KT), 1)
    vmin = jnp.full((_BM, _KT), jnp.inf, jnp.float32)
    varg = jnp.zeros((_BM, _KT), jnp.int32)
    for t in range(_K // _KT):
        et = e_ref[t * _KT:(t + 1) * _KT, :]
        mm = lax.dot_general(zb, et, (((1,), (1,)), ((), ())),
                             preferred_element_type=jnp.float32)
        d = (s1 - 2.0 * mm) + s2_ref[0, t * _KT:(t + 1) * _KT]
        cmp = d < vmin
        vmin = jnp.where(cmp, d, vmin)
        varg = jnp.where(cmp, tiota + t * _KT, varg)
    m = jnp.min(vmin, axis=1, keepdims=True)
    idx = jnp.min(jnp.where(vmin == m, varg, _K), axis=1)
    idx_ref[0, 0, :] = idx


def _tc_argmin(z_flat, e, s2t):
    n = z_flat.shape[0]
    grid = n // _BM
    out = pl.pallas_call(
        _tc_argmin_body,
        grid=(grid,),
        in_specs=[
            pl.BlockSpec((_BM, _C), lambda i: (i, 0)),
            pl.BlockSpec((_K, _C), lambda i: (0, 0)),
            pl.BlockSpec((1, _K), lambda i: (0, 0)),
        ],
        out_specs=pl.BlockSpec((1, 1, _BM), lambda i: (i, 0, 0)),
        out_shape=jax.ShapeDtypeStruct((grid, 1, _BM), jnp.int32),
    )(z_flat, e, s2t)
    return out.reshape(n)


_NW = 32
_CH = 64


def _sc_gather_body(table_hbm, idx_hbm, out_hbm, idx_v, rows_v, sem0, sem1):
    wid = lax.axis_index("s") * 2 + lax.axis_index("c")
    b_per_w = idx_v.shape[0]
    nch = b_per_w // _CH
    base = wid * b_per_w
    pltpu.sync_copy(idx_hbm.at[pl.ds(base, b_per_w)], idx_v)
    sems = (sem0, sem1)

    def start(c):
        return pltpu.async_copy(
            table_hbm.at[idx_v.at[pl.ds(c * _CH, _CH)]],
            rows_v.at[c % 2], sems[c % 2])

    cp = start(0)
    for c in range(nch):
        nxt = start(c + 1) if c + 1 < nch else None
        cp.wait()
        pltpu.sync_copy(rows_v.at[c % 2],
                        out_hbm.at[pl.ds(base + c * _CH, _CH)])
        cp = nxt


def _sc_gather(table, idx):
    n = idx.shape[0]
    b_per_w = n // _NW
    mesh = plsc.VectorSubcoreMesh(core_axis_name="c", subcore_axis_name="s")
    f = functools.partial(
        pl.kernel,
        out_type=jax.ShapeDtypeStruct((n, _C), jnp.float32),
        mesh=mesh,
        scratch_types=[
            pltpu.VMEM((b_per_w,), jnp.int32),
            pltpu.VMEM((2, _CH, _C), jnp.float32),
            pltpu.SemaphoreType.DMA,
            pltpu.SemaphoreType.DMA,
        ],
    )(_sc_gather_body)
    return f(table, idx)


_NCHUNK = 4


def kernel(z, embedding_weight):
    B, C, H, W = z.shape
    z_flat = jnp.transpose(z, (0, 2, 3, 1)).reshape(-1, C)
    s2t = jnp.sum(embedding_weight ** 2, axis=1, keepdims=True).T
    bc = B // _NCHUNK
    rows = bc * H * W
    quants, idxs = [], []
    for ci in range(_NCHUNK):
        zc = lax.slice_in_dim(z_flat, ci * rows, (ci + 1) * rows, axis=0)
        idx_c = _tc_argmin(zc, embedding_weight, s2t)
        g_c = _sc_gather(embedding_weight, idx_c)
        gv = g_c.reshape(bc, C, 2, C)
        v = jnp.transpose(gv, (0, 2, 3, 1))
        v = v.reshape(bc, H, W, C)
        quants.append(jnp.transpose(v, (0, 3, 1, 2)))
        idxs.append(idx_c.reshape(bc, -1))
    quantized = jnp.concatenate(quants, axis=0)
    idx = jnp.concatenate(idxs, axis=0)
    return (quantized, idx)
